# baseline (speedup 1.0000x reference)
import functools
import jax
import jax.numpy as jnp
from jax import lax
from jax.experimental import pallas as pl
from jax.experimental.pallas import tpu as pltpu
from jax.experimental.pallas import tpu_sc as plsc

_NC, _NS, _L = 2, 16, 16
_NW = _NC * _NS
_C = 32


def kernel(x, pos_embedding):
    batch, seq, dim = x.shape
    seq_per_w = seq // _NW
    n_chunks = seq_per_w // _C
    n_vec = dim // _L
    mesh = plsc.VectorSubcoreMesh(core_axis_name="c", subcore_axis_name="s")

    @functools.partial(
        pl.kernel,
        mesh=mesh,
        out_type=jax.ShapeDtypeStruct((batch, seq, dim), jnp.float32),
        scratch_types=[
            pltpu.VMEM((_C, dim), jnp.float32),
            pltpu.VMEM((_C, dim), jnp.float32),
        ],
    )
    def k(x_hbm, pos_hbm, out_hbm, pos_buf, x_buf):
        wid = lax.axis_index("s") * _NC + lax.axis_index("c")
        s0 = wid * seq_per_w

        def chunk_body(ci, carry):
            srow = s0 + ci * _C
            pltpu.sync_copy(pos_hbm.at[pl.ds(srow, _C)], pos_buf)
            for b in range(batch):
                pltpu.sync_copy(x_hbm.at[b, pl.ds(srow, _C)], x_buf)

                def add_row(r, c2):
                    for v in range(n_vec):
                        sl = pl.ds(v * _L, _L)
                        plsc.addupdate(x_buf.at[r, sl], pos_buf[r, sl])
                    return c2

                lax.fori_loop(0, _C, add_row, 0)
                pltpu.sync_copy(x_buf, out_hbm.at[b, pl.ds(srow, _C)])
            return carry

        lax.fori_loop(0, n_chunks, chunk_body, 0)

    return k(x, pos_embedding)


# speedup vs baseline: 2.1631x; 2.1631x over previous
import functools
import jax
import jax.numpy as jnp
from jax import lax
from jax.experimental import pallas as pl
from jax.experimental.pallas import tpu as pltpu
from jax.experimental.pallas import tpu_sc as plsc

_NC, _NS, _L = 2, 16, 16
_NW = _NC * _NS
_C = 4
_NSET = 4


def kernel(x, pos_embedding):
    batch, seq, dim = x.shape
    seq_per_w = seq // _NW
    n_chunks = seq_per_w // _C
    n_outer = n_chunks // _NSET
    n_vec = dim // _L
    mesh = plsc.VectorSubcoreMesh(core_axis_name="c", subcore_axis_name="s")

    @functools.partial(
        pl.kernel,
        mesh=mesh,
        out_type=jax.ShapeDtypeStruct((batch, seq, dim), jnp.float32),
        scratch_types=(
            [pltpu.VMEM((batch, _C, dim), jnp.float32) for _ in range(_NSET)]
            + [pltpu.VMEM((_C, dim), jnp.float32) for _ in range(_NSET)]
            + [pltpu.SemaphoreType.DMA for _ in range(2 * _NSET)]
        ),
    )
    def k(x_hbm, pos_hbm, out_hbm, xb0, xb1, xb2, xb3, pb0, pb1, pb2, pb3,
          si0, si1, si2, si3, so0, so1, so2, so3):
        xbs = (xb0, xb1, xb2, xb3)
        pbs = (pb0, pb1, pb2, pb3)
        sis = (si0, si1, si2, si3)
        sos = (so0, so1, so2, so3)
        wid = lax.axis_index("s") * _NC + lax.axis_index("c")
        s0 = wid * seq_per_w

        def xin(ci, b, st):
            return pltpu.make_async_copy(
                x_hbm.at[b, pl.ds(s0 + ci * _C, _C)], xbs[st].at[b], sis[st])

        def pin(ci, st):
            return pltpu.make_async_copy(
                pos_hbm.at[pl.ds(s0 + ci * _C, _C)], pbs[st], sis[st])

        def xout(ci, b, st):
            return pltpu.make_async_copy(
                xbs[st].at[b], out_hbm.at[b, pl.ds(s0 + ci * _C, _C)],
                sos[st])

        def fire(ci, st):
            pin(ci, st).start()
            for b in range(batch):
                xin(ci, b, st).start()

        def process(ci, st):
            pin(ci, st).wait()
            for b in range(batch):
                xin(ci, b, st).wait()
            for b in range(batch):
                def add_row(r, c):
                    for v in range(n_vec):
                        sl = pl.ds(v * _L, _L)
                        plsc.addupdate(xbs[st].at[b, r, sl], pbs[st][r, sl])
                    return c

                lax.fori_loop(0, _C, add_row, 0)
                xout(ci, b, st).start()

        def drain(ci, st):
            for b in range(batch):
                xout(ci, b, st).wait()

        fire(0, 0)
        fire(1, 1)

        def outer(i, carry):
            for st in range(_NSET):
                ci = i * _NSET + st
                tgt = (st + 2) % _NSET
                process(ci, st)
                if st < 2:
                    @pl.when(i >= 1)
                    def _():
                        drain(ci - 2, tgt)

                    fire(ci + 2, tgt)
                else:
                    @pl.when(i < n_outer - 1)
                    def _():
                        drain(ci - 2, tgt)
                        fire(ci + 2, tgt)
            return carry

        lax.fori_loop(0, n_outer, outer, 0)
        for st in range(_NSET):
            drain(n_chunks - _NSET + st, st)

    return k(x, pos_embedding)


# speedup vs baseline: 2.4670x; 1.1405x over previous
import functools
import jax
import jax.numpy as jnp
from jax import lax
from jax.experimental import pallas as pl
from jax.experimental.pallas import tpu as pltpu
from jax.experimental.pallas import tpu_sc as plsc

_NC, _NS, _L = 2, 16, 16
_NW = _NC * _NS
_C = 8
_NSET = 3
_NPOS = 2
_UNROLL = _NSET * _NPOS


def kernel(x, pos_embedding):
    batch, seq, dim = x.shape
    seq_per_w = seq // _NW
    n_chunks = seq_per_w // _C
    n_outer = (n_chunks - 2) // _UNROLL
    n_vec = dim // _L
    mesh = plsc.VectorSubcoreMesh(core_axis_name="c", subcore_axis_name="s")

    @functools.partial(
        pl.kernel,
        mesh=mesh,
        out_type=jax.ShapeDtypeStruct((batch, seq, dim), jnp.float32),
        scratch_types=(
            [pltpu.VMEM((batch, _C, dim), jnp.float32) for _ in range(_NSET)]
            + [pltpu.VMEM((_C, dim), jnp.float32) for _ in range(_NPOS)]
            + [pltpu.SemaphoreType.DMA for _ in range(2 * _NSET + _NPOS)]
        ),
    )
    def k(x_hbm, pos_hbm, out_hbm, xb0, xb1, xb2, pb0, pb1,
          si0, si1, si2, so0, so1, so2, sp0, sp1):
        xbs = (xb0, xb1, xb2)
        pbs = (pb0, pb1)
        sis = (si0, si1, si2)
        sos = (so0, so1, so2)
        sps = (sp0, sp1)
        wid = lax.axis_index("s") * _NC + lax.axis_index("c")
        s0 = wid * seq_per_w

        def xin(ci, st):
            return pltpu.make_async_copy(
                x_hbm.at[pl.ds(0, batch), pl.ds(s0 + ci * _C, _C)],
                xbs[st], sis[st])

        def pin(ci, pp):
            return pltpu.make_async_copy(
                pos_hbm.at[pl.ds(s0 + ci * _C, _C)], pbs[pp], sps[pp])

        def xout(ci, st):
            return pltpu.make_async_copy(
                xbs[st], out_hbm.at[pl.ds(0, batch), pl.ds(s0 + ci * _C, _C)],
                sos[st])

        def fire(ci, st, pp):
            pin(ci, pp).start()
            xin(ci, st).start()

        def compute(ci, st, pp):
            @plsc.parallel_loop(0, batch * _C, unroll=1)
            def _(i):
                b = i // _C
                r = i % _C
                for v in range(n_vec):
                    sl = pl.ds(v * _L, _L)
                    plsc.addupdate(xbs[st].at[b, r, sl], pbs[pp][r, sl])

            xout(ci, st).start()

        def process(ci, st, pp):
            pin(ci, pp).wait()
            xin(ci, st).wait()
            compute(ci, st, pp)

        fire(0, 0, 0)
        fire(1, 1, 1)

        def outer(j, carry):
            for k_ in range(_UNROLL):
                ci = j * _UNROLL + k_
                st = k_ % _NSET
                pp = k_ % _NPOS
                tgt = (st + 2) % _NSET
                pin(ci, pp).wait()
                xin(ci, st).wait()
                if k_ == 0:
                    @pl.when(j >= 1)
                    def _():
                        xout(ci - 1, tgt).wait()
                else:
                    xout(ci - 1, tgt).wait()
                xin(ci + 2, tgt).start()
                compute(ci, st, pp)
                pin(ci + 2, pp).start()
            return carry

        lax.fori_loop(0, n_outer, outer, 0)
        c2, c1 = n_chunks - 2, n_chunks - 1
        process(c2, c2 % _NSET, c2 % _NPOS)
        process(c1, c1 % _NSET, c1 % _NPOS)
        for cj in (n_chunks - 3, c2, c1):
            xout(cj, cj % _NSET).wait()

    return k(x, pos_embedding)
